# baseline (speedup 1.0000x reference)
import functools

import jax
import jax.numpy as jnp
from jax import lax
from jax.experimental import pallas as pl
from jax.experimental.pallas import tpu as pltpu
from jax.experimental.pallas import tpu_sc as plsc

_B = 16384
_D = 32
_NC = 2
_NS = 16
_NW = _NC * _NS
_BPW = _B // _NW
_CH = 16
_NCH = _BPW // _CH


def _cf_body(uidx_hbm, iidx_hbm, utab_hbm, itab_hbm, ubias_hbm, ibias_hbm,
             out_hbm, uidx_v, iidx_v, urows_v, irows_v, ub_v, ib_v, out_v,
             sem_u, sem_i, sem_bu, sem_bi):
    wid = lax.axis_index("s") * _NC + lax.axis_index("c")
    base = wid * _BPW

    pltpu.sync_copy(uidx_hbm.at[pl.ds(base, _BPW)], uidx_v)
    pltpu.sync_copy(iidx_hbm.at[pl.ds(base, _BPW)], iidx_v)

    cu = pltpu.async_copy(utab_hbm.at[uidx_v], urows_v, sem_u)
    ci = pltpu.async_copy(itab_hbm.at[iidx_v], irows_v, sem_i)
    cbu = pltpu.async_copy(ubias_hbm.at[uidx_v], ub_v, sem_bu)
    cbi = pltpu.async_copy(ibias_hbm.at[iidx_v], ib_v, sem_bi)
    cu.wait()
    ci.wait()
    cbu.wait()
    cbi.wait()

    lane = lax.iota(jnp.int32, 16)
    zero16 = jnp.zeros((16,), jnp.int32)

    def chunk(c, _):
        rows = lane + c * _CH
        acc = ub_v[pl.ds(c * _CH, _CH)] + ib_v[pl.ds(c * _CH, _CH)]
        for d in range(_D):
            col = jnp.full((16,), d, jnp.int32)
            uc = plsc.load_gather(urows_v, [rows, col])
            ic = plsc.load_gather(irows_v, [rows, col])
            acc = acc + uc * ic
        out_v[pl.ds(c * _CH, _CH)] = acc
        return _

    lax.fori_loop(0, _NCH, chunk, None)
    pltpu.sync_copy(out_v, out_hbm.at[pl.ds(base, _BPW)])


@jax.jit
def _cf_predict(user_indices, item_indices, user_emb_table, item_emb_table,
                user_bias_table, item_bias_table):
    mesh = plsc.VectorSubcoreMesh(core_axis_name="c", subcore_axis_name="s")
    f = pl.kernel(
        _cf_body,
        out_type=jax.ShapeDtypeStruct((_B,), jnp.float32),
        mesh=mesh,
        scratch_types=[
            pltpu.VMEM((_BPW,), jnp.int32),
            pltpu.VMEM((_BPW,), jnp.int32),
            pltpu.VMEM((_BPW, _D), jnp.float32),
            pltpu.VMEM((_BPW, _D), jnp.float32),
            pltpu.VMEM((_BPW,), jnp.float32),
            pltpu.VMEM((_BPW,), jnp.float32),
            pltpu.VMEM((_BPW,), jnp.float32),
            pltpu.SemaphoreType.DMA,
            pltpu.SemaphoreType.DMA,
            pltpu.SemaphoreType.DMA,
            pltpu.SemaphoreType.DMA,
        ],
        compiler_params=pltpu.CompilerParams(
            needs_layout_passes=False, use_tc_tiling_on_sc=False),
    )
    return f(user_indices, item_indices, user_emb_table, item_emb_table,
             user_bias_table, item_bias_table)


def kernel(user_indices, item_indices, user_emb_table, item_emb_table,
           user_bias_table, item_bias_table):
    return _cf_predict(user_indices, item_indices, user_emb_table,
                       item_emb_table, user_bias_table.reshape(-1),
                       item_bias_table.reshape(-1))
